# baseline (speedup 1.0000x reference)
import jax
import jax.numpy as jnp
from jax.experimental import pallas as pl
from jax.experimental.pallas import tpu as pltpu

NUM_EXPERTS = 8
TOP_K = 2
SEQ = 2048
D_MODEL = 1024
MLP_DIM = 4096

BT = 256
FB = 1024
NF = MLP_DIM // FB
NB = (SEQ * TOP_K) // BT + NUM_EXPERTS
R = NB * BT
NA = SEQ * TOP_K
NTB = SEQ // BT


def _moe_mlp_kernel(s_ref, tw_ref, x_ref, w0_ref, w1_ref, wo_ref, o_ref,
                    acc_ref):
    j = pl.program_id(0)
    i = pl.program_id(1)
    nb = s_ref[NB]

    @pl.when(jnp.logical_and(j < NF, i < nb))
    def _():
        x = x_ref[pl.ds(i * BT, BT), :]
        h0 = jnp.dot(x, w0_ref[0], preferred_element_type=jnp.float32)
        h1 = jnp.dot(x, w1_ref[0], preferred_element_type=jnp.float32)
        h = jax.nn.silu(h0) * h1
        y = jnp.dot(h, wo_ref[0], preferred_element_type=jnp.float32)

        @pl.when(j == 0)
        def _():
            acc_ref[pl.ds(i * BT, BT), :] = y

        @pl.when(j > 0)
        def _():
            acc_ref[pl.ds(i * BT, BT), :] += y

    @pl.when(jnp.logical_and(j == NF, i < NTB))
    def _():
        base = i * BT
        for rb in range(BT // 8):
            rows = []
            for u in range(8):
                t = base + rb * 8 + u
                r0 = acc_ref[pl.ds(s_ref[NB + 1 + 2 * t], 1), :]
                r1 = acc_ref[pl.ds(s_ref[NB + 1 + 2 * t + 1], 1), :]
                rows.append(tw_ref[2 * t] * r0 + tw_ref[2 * t + 1] * r1)
            o_ref[rb * 8:(rb + 1) * 8, :] = jnp.concatenate(rows, axis=0)


def _grouped_mlp(meta, tw, x_g, wi_0, wi_1, wo):
    grid_spec = pltpu.PrefetchScalarGridSpec(
        num_scalar_prefetch=2,
        grid=(NF + 1, NB),
        in_specs=[
            pl.BlockSpec((R, D_MODEL), lambda j, i, s, w: (0, 0)),
            pl.BlockSpec((1, D_MODEL, FB),
                         lambda j, i, s, w: (jnp.where(j == NF,
                                                    NUM_EXPERTS - 1, s[i]),
                                          0, jnp.minimum(j, NF - 1))),
            pl.BlockSpec((1, D_MODEL, FB),
                         lambda j, i, s, w: (jnp.where(j == NF,
                                                    NUM_EXPERTS - 1, s[i]),
                                          0, jnp.minimum(j, NF - 1))),
            pl.BlockSpec((1, FB, D_MODEL),
                         lambda j, i, s, w: (jnp.where(j == NF,
                                                    NUM_EXPERTS - 1, s[i]),
                                          jnp.minimum(j, NF - 1), 0)),
        ],
        out_specs=pl.BlockSpec(
            (BT, D_MODEL),
            lambda j, i, s, w: (jnp.where(j == NF,
                                       jnp.minimum(i, NTB - 1), 0), 0)),
        scratch_shapes=[pltpu.VMEM((R, D_MODEL), jnp.float32)],
    )
    return pl.pallas_call(
        _moe_mlp_kernel,
        grid_spec=grid_spec,
        out_shape=jax.ShapeDtypeStruct((SEQ, D_MODEL), jnp.float32),
        compiler_params=pltpu.CompilerParams(
            dimension_semantics=("arbitrary", "arbitrary"),
            vmem_limit_bytes=67000000,
        ),
    )(meta, tw, x_g, wi_0, wi_1, wo)


def kernel(inputs, gate_w, wi_0, wi_1, wo):
    x = inputs.reshape(SEQ, D_MODEL)

    logits = x @ gate_w
    e0 = jnp.argmax(logits, axis=-1).astype(jnp.int32)
    v0 = jnp.max(logits, axis=-1)
    masked = jnp.where(
        jax.nn.one_hot(e0, NUM_EXPERTS, dtype=jnp.bool_), -jnp.inf, logits)
    e1 = jnp.argmax(masked, axis=-1).astype(jnp.int32)
    v1 = jnp.max(masked, axis=-1)
    p1 = jax.nn.sigmoid(v1 - v0)
    top_w = jnp.stack([1.0 - p1, p1], axis=-1)
    experts_flat = jnp.stack([e0, e1], axis=-1).reshape(-1)

    onehot = (experts_flat[:, None] ==
              jnp.arange(NUM_EXPERTS)[None, :]).astype(jnp.int32)
    csum = jnp.cumsum(onehot, axis=0)
    counts = csum[-1]
    ranks = jnp.take_along_axis(csum, experts_flat[:, None], axis=1)[:, 0] - 1
    padded_counts = ((counts + BT - 1) // BT) * BT
    padded_offsets = jnp.concatenate(
        [jnp.zeros((1,), jnp.int32), jnp.cumsum(padded_counts)[:-1]]
    ).astype(jnp.int32)
    pos = padded_offsets[experts_flat] + ranks
    num_blocks = (padded_offsets[-1] + padded_counts[-1]) // BT

    token_of = jnp.arange(NA, dtype=jnp.int32) // TOP_K
    gather_idx = jnp.zeros((R,), jnp.int32).at[pos].set(
        token_of, unique_indices=True, mode="promise_in_bounds")
    block_expert = (
        jnp.searchsorted(padded_offsets,
                         jnp.arange(NB, dtype=jnp.int32) * BT, side="right")
        - 1
    ).astype(jnp.int32)
    meta = jnp.concatenate(
        [block_expert, num_blocks.reshape(1).astype(jnp.int32), pos])

    x_g = x.astype(jnp.bfloat16)[gather_idx]

    out = _grouped_mlp(meta, top_w.reshape(-1), x_g, wi_0, wi_1, wo)
    return out.reshape(1, SEQ, D_MODEL)
